# baseline (speedup 1.0000x reference)
import functools

import jax
import jax.numpy as jnp
from jax import lax
from jax.experimental import pallas as pl
from jax.experimental.pallas import tpu as pltpu
from jax.experimental.pallas import tpu_sc as plsc

N = 10000
E = 320000
NODE_IN = 128
EDGE_IN = 16
MSG = 32
NODE_OUT = 128
EDGE_OUT = 16

NC = 2
NS = 16
NW = NC * NS
EPW = E // NW
C = 80
CPW = EPW // C
N_ACC = 10240
NPS = N_ACC // NS
CNTW = 16

f32 = jnp.float32

@functools.cache
def _sc_kernels():
    mesh = plsc.VectorSubcoreMesh(
        core_axis_name="c", subcore_axis_name="s", num_cores=NC, num_subcores=NS
    )

    @functools.partial(
        pl.kernel,
        out_type=[
            jax.ShapeDtypeStruct((MSG, E), f32),
            jax.ShapeDtypeStruct((NC, N_ACC, CNTW), f32),
        ],
        mesh=mesh,
        compiler_params=pltpu.CompilerParams(use_tc_tiling_on_sc=False, needs_layout_passes=False),
        scratch_types=[
            pltpu.VMEM((CPW, C), jnp.int32),
            pltpu.VMEM((CPW, C), jnp.int32),
            pltpu.VMEM((C, MSG), f32),
            pltpu.VMEM((C, MSG), f32),
            pltpu.VMEM((MSG, C), f32),
            pltpu.VMEM((C, CNTW), f32),
            pltpu.VMEM((NPS, CNTW), f32),
            pltpu.VMEM_SHARED((N_ACC, CNTW), f32),
            pltpu.SemaphoreType.DMA,
            pltpu.SemaphoreType.DMA,
        ],
    )
    def sc_gather(pj, pi, srcm, dstm, gT, cntp, idx_s, idx_d, rows_s, rows_d,
                  sum_t, ones_v, stage, cnt_sh, sem1, sem2):
        cid = lax.axis_index("c")
        sid = lax.axis_index("s")
        wid = sid * NC + cid

        def _fill(i, _):
            stage[i, pl.ds(0, 16)] = jnp.zeros((16,), f32)
            return 0

        lax.fori_loop(0, NPS, _fill, 0)

        def _fill1(i, _):
            ones_v[i, pl.ds(0, 16)] = jnp.ones((16,), f32)
            return 0

        lax.fori_loop(0, C, _fill1, 0)

        pltpu.sync_copy(stage, cnt_sh.at[pl.ds(sid * NPS, NPS)])
        plsc.subcore_barrier()

        pltpu.sync_copy(srcm.at[wid], idx_s)
        pltpu.sync_copy(dstm.at[wid], idx_d)

        i16 = lax.iota(jnp.int32, 16)

        def _chunk(i, _):
            base = wid * EPW + i * C
            cp1 = pltpu.async_copy(pj.at[idx_s.at[i]], rows_s, sem1)
            cp2 = pltpu.async_copy(pi.at[idx_d.at[i]], rows_d, sem2)
            cp1.wait()
            cp2.wait()

            for f in range(MSG):
                fdx = jnp.full((16,), f, jnp.int32)
                for b in range(C // 16):
                    jdx = i16 + (16 * b)
                    v = (plsc.load_gather(rows_s, [jdx, fdx])
                         + plsc.load_gather(rows_d, [jdx, fdx]))
                    sum_t[f, pl.ds(16 * b, 16)] = v
            pltpu.sync_copy(sum_t, gT.at[:, pl.ds(base, C)])
            pltpu.sync_copy(ones_v, cnt_sh.at[idx_d.at[i]], add=True)
            return 0

        lax.fori_loop(0, CPW, _chunk, 0)
        plsc.subcore_barrier()

        pltpu.sync_copy(cnt_sh.at[pl.ds(sid * NPS, NPS)], stage)
        pltpu.sync_copy(stage, cntp.at[cid].at[pl.ds(sid * NPS, NPS)])

    @functools.partial(
        pl.kernel,
        out_type=jax.ShapeDtypeStruct((NC, N_ACC, MSG), f32),
        mesh=mesh,
        compiler_params=pltpu.CompilerParams(use_tc_tiling_on_sc=False, needs_layout_passes=False),
        scratch_types=[
            pltpu.VMEM((CPW, C), jnp.int32),
            pltpu.VMEM((MSG, C), f32),
            pltpu.VMEM((C, MSG), f32),
            pltpu.VMEM((NPS, MSG), f32),
            pltpu.VMEM_SHARED((N_ACC, MSG), f32),
            pltpu.SemaphoreType.DMA,
        ],
    )
    def sc_scatter(msgT, dstm, sump, idx_d, msg_t, rows_m, stage, acc_sh, sem):
        cid = lax.axis_index("c")
        sid = lax.axis_index("s")
        wid = sid * NC + cid

        def _fill(i, _):
            stage[i, pl.ds(0, 16)] = jnp.zeros((16,), f32)
            stage[i, pl.ds(16, 16)] = jnp.zeros((16,), f32)
            return 0

        lax.fori_loop(0, NPS, _fill, 0)
        pltpu.sync_copy(stage, acc_sh.at[pl.ds(sid * NPS, NPS)])
        plsc.subcore_barrier()

        pltpu.sync_copy(dstm.at[wid], idx_d)

        i16 = lax.iota(jnp.int32, 16)

        def _chunk(i, _):
            base = wid * EPW + i * C
            pltpu.sync_copy(msgT.at[:, pl.ds(base, C)], msg_t)

            for f in range(MSG):
                fdx = jnp.full((16,), f, jnp.int32)
                for b in range(C // 16):
                    jdx = i16 + (16 * b)
                    v = msg_t[f, pl.ds(16 * b, 16)]
                    plsc.store_scatter(rows_m, [jdx, fdx], v)
            pltpu.sync_copy(rows_m, acc_sh.at[idx_d.at[i]], add=True)
            return 0

        lax.fori_loop(0, CPW, _chunk, 0)
        plsc.subcore_barrier()

        pltpu.sync_copy(acc_sh.at[pl.ds(sid * NPS, NPS)], stage)
        pltpu.sync_copy(stage, sump.at[cid].at[pl.ds(sid * NPS, NPS)])

    return sc_gather, sc_scatter


_RN = 1000
_RE = 2000


def _proj_body(x_ref, w_ref, pj_ref, pi_ref, xw_ref):
    acc = jnp.dot(x_ref[...], w_ref[...], preferred_element_type=f32)
    pj_ref[...] = acc[:, :MSG]
    pi_ref[...] = acc[:, MSG:2 * MSG]
    xw_ref[...] = acc[:, 2 * MSG:]


def _edge_body(g_ref, ea_ref, r_ref, di_ref, ws_ref, wea_ref, bm_ref, we_ref,
               be_ref, msg_ref, eo_ref, st_ref):
    r = r_ref[...]
    d = jnp.sqrt(jnp.sum(r * r, axis=0, keepdims=True))
    di = di_ref[...]
    st = (d - di) / di
    pre = (g_ref[...] + ws_ref[...] * st + bm_ref[...]
           + jnp.dot(wea_ref[...], ea_ref[...], preferred_element_type=f32))
    m = jnp.maximum(pre, 0.0) + jnp.log1p(jnp.exp(-jnp.abs(pre)))
    msg_ref[...] = m
    eo_ref[...] = jnp.dot(we_ref[...], m, preferred_element_type=f32) + be_ref[...]
    st_ref[...] = st


def _upd_body(xw_ref, sp_ref, cp_ref, w2_ref, bu_ref, out_ref):
    cnt = cp_ref[0, :, 0:1] + cp_ref[1, :, 0:1]
    aggr = (sp_ref[0] + sp_ref[1]) / jnp.clip(cnt, 1.0, None)
    out_ref[...] = (xw_ref[...] + bu_ref[...]
                    + jnp.dot(aggr, w2_ref[...], preferred_element_type=f32))


def _full(shape):
    return pl.BlockSpec(shape, lambda i: tuple(0 for _ in shape))


_proj_call = pl.pallas_call(
    _proj_body,
    grid=(N // _RN,),
    in_specs=[
        pl.BlockSpec((_RN, NODE_IN), lambda i: (i, 0)),
        _full((NODE_IN, 2 * MSG + NODE_OUT)),
    ],
    out_specs=[
        pl.BlockSpec((_RN, MSG), lambda i: (i, 0)),
        pl.BlockSpec((_RN, MSG), lambda i: (i, 0)),
        pl.BlockSpec((_RN, NODE_OUT), lambda i: (i, 0)),
    ],
    out_shape=[
        jax.ShapeDtypeStruct((N, MSG), f32),
        jax.ShapeDtypeStruct((N, MSG), f32),
        jax.ShapeDtypeStruct((N, NODE_OUT), f32),
    ],
)

_EC = 6400

_edge_call = pl.pallas_call(
    _edge_body,
    grid=(E // _EC,),
    in_specs=[
        pl.BlockSpec((MSG, _EC), lambda i: (0, i)),
        pl.BlockSpec((EDGE_IN, _EC), lambda i: (0, i)),
        pl.BlockSpec((3, _EC), lambda i: (0, i)),
        pl.BlockSpec((1, _EC), lambda i: (0, i)),
        _full((MSG, 1)),
        _full((MSG, EDGE_IN)),
        _full((MSG, 1)),
        _full((EDGE_OUT, MSG)),
        _full((EDGE_OUT, 1)),
    ],
    out_specs=[
        pl.BlockSpec((MSG, _EC), lambda i: (0, i)),
        pl.BlockSpec((EDGE_OUT, _EC), lambda i: (0, i)),
        pl.BlockSpec((1, _EC), lambda i: (0, i)),
    ],
    out_shape=[
        jax.ShapeDtypeStruct((MSG, E), f32),
        jax.ShapeDtypeStruct((EDGE_OUT, E), f32),
        jax.ShapeDtypeStruct((1, E), f32),
    ],
)

_upd_call = pl.pallas_call(
    _upd_body,
    grid=(N // _RN,),
    in_specs=[
        pl.BlockSpec((_RN, NODE_OUT), lambda i: (i, 0)),
        pl.BlockSpec((NC, _RN, MSG), lambda i: (0, i, 0)),
        pl.BlockSpec((NC, _RN, CNTW), lambda i: (0, i, 0)),
        _full((MSG, NODE_OUT)),
        _full((1, NODE_OUT)),
    ],
    out_specs=pl.BlockSpec((_RN, NODE_OUT), lambda i: (i, 0)),
    out_shape=jax.ShapeDtypeStruct((N, NODE_OUT), f32),
)


def kernel(x, edge_index, edge_attr, r, d_init, W_msg, b_msg, W_upd, b_upd,
           W_edge, b_edge):
    srcm = edge_index[0].reshape(NW, CPW, C)
    dstm = edge_index[1].reshape(NW, CPW, C)
    wcat = jnp.concatenate(
        [W_msg[:NODE_IN], W_msg[NODE_IN:2 * NODE_IN], W_upd[:NODE_IN]], axis=1)

    sc_gather, sc_scatter = _sc_kernels()
    pj, pi, xw1 = _proj_call(x, wcat)
    gT, cntp = sc_gather(pj, pi, srcm, dstm)
    msgsT, eoT, stT = _edge_call(
        gT, edge_attr.T, r.T, d_init.T,
        W_msg[2 * NODE_IN:2 * NODE_IN + 1].T, W_msg[2 * NODE_IN + 1:].T,
        b_msg.reshape(1, MSG).T, W_edge.T, b_edge.reshape(1, EDGE_OUT).T)
    sump = sc_scatter(msgsT, dstm)
    x_new = _upd_call(xw1, sump, cntp, W_upd[NODE_IN:], b_upd.reshape(1, NODE_OUT))
    return (x_new, eoT.T, msgsT.T, stT.T)


# speedup vs baseline: 1.1000x; 1.1000x over previous
import functools

import jax
import jax.numpy as jnp
from jax import lax
from jax.experimental import pallas as pl
from jax.experimental.pallas import tpu as pltpu
from jax.experimental.pallas import tpu_sc as plsc

N = 10000
E = 320000
NODE_IN = 128
EDGE_IN = 16
MSG = 32
NODE_OUT = 128
EDGE_OUT = 16

NC = 2
NS = 16
NW = NC * NS
EPW = E // NW
C = 80
CPW = EPW // C
SUB = 5
CIO = C * SUB
N_ACC = 10240
NPS = N_ACC // NS
CNTW = 16

f32 = jnp.float32

@functools.cache
def _sc_kernels():
    mesh = plsc.VectorSubcoreMesh(
        core_axis_name="c", subcore_axis_name="s", num_cores=NC, num_subcores=NS
    )

    @functools.partial(
        pl.kernel,
        out_type=[
            jax.ShapeDtypeStruct((MSG, E), f32),
            jax.ShapeDtypeStruct((NC, N_ACC, CNTW), f32),
        ],
        mesh=mesh,
        compiler_params=pltpu.CompilerParams(use_tc_tiling_on_sc=False, needs_layout_passes=False),
        scratch_types=[
            pltpu.VMEM((CPW, C), jnp.int32),
            pltpu.VMEM((CPW, C), jnp.int32),
            pltpu.VMEM((SUB, C, MSG), f32),
            pltpu.VMEM((SUB, C, MSG), f32),
            pltpu.VMEM((MSG, CIO), f32),
            pltpu.VMEM((C, CNTW), f32),
            pltpu.VMEM((NPS, CNTW), f32),
            pltpu.VMEM_SHARED((N_ACC, CNTW), f32),
            pltpu.SemaphoreType.DMA,
            pltpu.SemaphoreType.DMA,
        ],
    )
    def sc_gather(pj, pi, srcm, dstm, gT, cntp, idx_s, idx_d, rows_s, rows_d,
                  sum_t, ones_v, stage, cnt_sh, sem1, sem2):
        cid = lax.axis_index("c")
        sid = lax.axis_index("s")
        wid = sid * NC + cid

        def _fill(i, _):
            stage[i, pl.ds(0, 16)] = jnp.zeros((16,), f32)
            return 0

        lax.fori_loop(0, NPS, _fill, 0)

        def _fill1(i, _):
            ones_v[i, pl.ds(0, 16)] = jnp.ones((16,), f32)
            return 0

        lax.fori_loop(0, C, _fill1, 0)

        pltpu.sync_copy(stage, cnt_sh.at[pl.ds(sid * NPS, NPS)])
        plsc.subcore_barrier()

        pltpu.sync_copy(srcm.at[wid], idx_s)
        pltpu.sync_copy(dstm.at[wid], idx_d)

        i16 = lax.iota(jnp.int32, 16)

        def _chunk(i, _):
            base = wid * EPW + i * CIO
            cps = []
            for s in range(SUB):
                cps.append(pltpu.async_copy(
                    pj.at[idx_s.at[SUB * i + s]], rows_s.at[s], sem1))
                cps.append(pltpu.async_copy(
                    pi.at[idx_d.at[SUB * i + s]], rows_d.at[s], sem2))
            for s in range(SUB):
                cps[2 * s].wait()
                cps[2 * s + 1].wait()
                for f in range(MSG):
                    fdx = jnp.full((16,), f, jnp.int32)
                    for b in range(C // 16):
                        jdx = i16 + (16 * b)
                        v = (plsc.load_gather(rows_s.at[s], [jdx, fdx])
                             + plsc.load_gather(rows_d.at[s], [jdx, fdx]))
                        sum_t[f, pl.ds(C * s + 16 * b, 16)] = v
            pltpu.sync_copy(sum_t, gT.at[:, pl.ds(base, CIO)])
            for s in range(SUB):
                pltpu.sync_copy(ones_v, cnt_sh.at[idx_d.at[SUB * i + s]], add=True)
            return 0

        lax.fori_loop(0, CPW // SUB, _chunk, 0)
        plsc.subcore_barrier()

        pltpu.sync_copy(cnt_sh.at[pl.ds(sid * NPS, NPS)], stage)
        pltpu.sync_copy(stage, cntp.at[cid].at[pl.ds(sid * NPS, NPS)])

    @functools.partial(
        pl.kernel,
        out_type=jax.ShapeDtypeStruct((NC, N_ACC, MSG), f32),
        mesh=mesh,
        compiler_params=pltpu.CompilerParams(use_tc_tiling_on_sc=False, needs_layout_passes=False),
        scratch_types=[
            pltpu.VMEM((CPW, C), jnp.int32),
            pltpu.VMEM((MSG, CIO), f32),
            pltpu.VMEM((SUB, C, MSG), f32),
            pltpu.VMEM((NPS, MSG), f32),
            pltpu.VMEM_SHARED((N_ACC, MSG), f32),
            pltpu.SemaphoreType.DMA,
        ],
    )
    def sc_scatter(msgT, dstm, sump, idx_d, msg_t, rows_m, stage, acc_sh, sem):
        cid = lax.axis_index("c")
        sid = lax.axis_index("s")
        wid = sid * NC + cid

        def _fill(i, _):
            stage[i, pl.ds(0, 16)] = jnp.zeros((16,), f32)
            stage[i, pl.ds(16, 16)] = jnp.zeros((16,), f32)
            return 0

        lax.fori_loop(0, NPS, _fill, 0)
        pltpu.sync_copy(stage, acc_sh.at[pl.ds(sid * NPS, NPS)])
        plsc.subcore_barrier()

        pltpu.sync_copy(dstm.at[wid], idx_d)

        i16 = lax.iota(jnp.int32, 16)

        def _chunk(i, _):
            base = wid * EPW + i * CIO
            pltpu.sync_copy(msgT.at[:, pl.ds(base, CIO)], msg_t)

            cps = []
            for s in range(SUB):
                for f in range(MSG):
                    fdx = jnp.full((16,), f, jnp.int32)
                    for b in range(C // 16):
                        jdx = i16 + (16 * b)
                        v = msg_t[f, pl.ds(C * s + 16 * b, 16)]
                        plsc.store_scatter(rows_m.at[s], [jdx, fdx], v)
            for s in range(SUB):
                cps.append(pltpu.async_copy(
                    rows_m.at[s], acc_sh.at[idx_d.at[SUB * i + s]], sem, add=True))
            for cp in cps:
                cp.wait()
            return 0

        lax.fori_loop(0, CPW // SUB, _chunk, 0)
        plsc.subcore_barrier()

        pltpu.sync_copy(acc_sh.at[pl.ds(sid * NPS, NPS)], stage)
        pltpu.sync_copy(stage, sump.at[cid].at[pl.ds(sid * NPS, NPS)])

    return sc_gather, sc_scatter


_RN = 1000
_RE = 2000


def _proj_body(x_ref, w_ref, pj_ref, pi_ref, xw_ref):
    acc = jnp.dot(x_ref[...], w_ref[...], preferred_element_type=f32)
    pj_ref[...] = acc[:, :MSG]
    pi_ref[...] = acc[:, MSG:2 * MSG]
    xw_ref[...] = acc[:, 2 * MSG:]


def _edge_body(g_ref, ea_ref, r_ref, di_ref, ws_ref, wea_ref, bm_ref, we_ref,
               be_ref, msg_ref, eo_ref, st_ref):
    r = r_ref[...]
    d = jnp.sqrt(jnp.sum(r * r, axis=0, keepdims=True))
    di = di_ref[...]
    st = (d - di) / di
    pre = (g_ref[...] + ws_ref[...] * st + bm_ref[...]
           + jnp.dot(wea_ref[...], ea_ref[...], preferred_element_type=f32))
    m = jnp.maximum(pre, 0.0) + jnp.log1p(jnp.exp(-jnp.abs(pre)))
    msg_ref[...] = m
    eo_ref[...] = jnp.dot(we_ref[...], m, preferred_element_type=f32) + be_ref[...]
    st_ref[...] = st


def _upd_body(xw_ref, sp_ref, cp_ref, w2_ref, bu_ref, out_ref):
    cnt = cp_ref[0, :, 0:1] + cp_ref[1, :, 0:1]
    aggr = (sp_ref[0] + sp_ref[1]) / jnp.clip(cnt, 1.0, None)
    out_ref[...] = (xw_ref[...] + bu_ref[...]
                    + jnp.dot(aggr, w2_ref[...], preferred_element_type=f32))


def _full(shape):
    return pl.BlockSpec(shape, lambda i: tuple(0 for _ in shape))


_proj_call = pl.pallas_call(
    _proj_body,
    grid=(N // _RN,),
    in_specs=[
        pl.BlockSpec((_RN, NODE_IN), lambda i: (i, 0)),
        _full((NODE_IN, 2 * MSG + NODE_OUT)),
    ],
    out_specs=[
        pl.BlockSpec((_RN, MSG), lambda i: (i, 0)),
        pl.BlockSpec((_RN, MSG), lambda i: (i, 0)),
        pl.BlockSpec((_RN, NODE_OUT), lambda i: (i, 0)),
    ],
    out_shape=[
        jax.ShapeDtypeStruct((N, MSG), f32),
        jax.ShapeDtypeStruct((N, MSG), f32),
        jax.ShapeDtypeStruct((N, NODE_OUT), f32),
    ],
)

_EC = 6400

_edge_call = pl.pallas_call(
    _edge_body,
    grid=(E // _EC,),
    in_specs=[
        pl.BlockSpec((MSG, _EC), lambda i: (0, i)),
        pl.BlockSpec((EDGE_IN, _EC), lambda i: (0, i)),
        pl.BlockSpec((3, _EC), lambda i: (0, i)),
        pl.BlockSpec((1, _EC), lambda i: (0, i)),
        _full((MSG, 1)),
        _full((MSG, EDGE_IN)),
        _full((MSG, 1)),
        _full((EDGE_OUT, MSG)),
        _full((EDGE_OUT, 1)),
    ],
    out_specs=[
        pl.BlockSpec((MSG, _EC), lambda i: (0, i)),
        pl.BlockSpec((EDGE_OUT, _EC), lambda i: (0, i)),
        pl.BlockSpec((1, _EC), lambda i: (0, i)),
    ],
    out_shape=[
        jax.ShapeDtypeStruct((MSG, E), f32),
        jax.ShapeDtypeStruct((EDGE_OUT, E), f32),
        jax.ShapeDtypeStruct((1, E), f32),
    ],
)

_upd_call = pl.pallas_call(
    _upd_body,
    grid=(N // _RN,),
    in_specs=[
        pl.BlockSpec((_RN, NODE_OUT), lambda i: (i, 0)),
        pl.BlockSpec((NC, _RN, MSG), lambda i: (0, i, 0)),
        pl.BlockSpec((NC, _RN, CNTW), lambda i: (0, i, 0)),
        _full((MSG, NODE_OUT)),
        _full((1, NODE_OUT)),
    ],
    out_specs=pl.BlockSpec((_RN, NODE_OUT), lambda i: (i, 0)),
    out_shape=jax.ShapeDtypeStruct((N, NODE_OUT), f32),
)


def kernel(x, edge_index, edge_attr, r, d_init, W_msg, b_msg, W_upd, b_upd,
           W_edge, b_edge):
    srcm = edge_index[0].reshape(NW, CPW, C)
    dstm = edge_index[1].reshape(NW, CPW, C)
    wcat = jnp.concatenate(
        [W_msg[:NODE_IN], W_msg[NODE_IN:2 * NODE_IN], W_upd[:NODE_IN]], axis=1)

    sc_gather, sc_scatter = _sc_kernels()
    pj, pi, xw1 = _proj_call(x, wcat)
    gT, cntp = sc_gather(pj, pi, srcm, dstm)
    msgsT, eoT, stT = _edge_call(
        gT, edge_attr.T, r.T, d_init.T,
        W_msg[2 * NODE_IN:2 * NODE_IN + 1].T, W_msg[2 * NODE_IN + 1:].T,
        b_msg.reshape(1, MSG).T, W_edge.T, b_edge.reshape(1, EDGE_OUT).T)
    sump = sc_scatter(msgsT, dstm)
    x_new = _upd_call(xw1, sump, cntp, W_upd[NODE_IN:], b_upd.reshape(1, NODE_OUT))
    return (x_new, eoT.T, msgsT.T, stT.T)
